# baseline (speedup 1.0000x reference)
import functools

import jax
import jax.numpy as jnp
from jax import lax
from jax.experimental import pallas as pl
from jax.experimental.pallas import tpu as pltpu
from jax.experimental.pallas import tpu_sc as plsc


@functools.lru_cache(maxsize=None)
def _build_gather(V, D, B, dtype):
    info = plsc.get_sparse_core_info()
    NC, NS = info.num_cores, info.num_subcores
    NW = NC * NS
    assert B % NW == 0
    b_per_w = B // NW
    CH = 256
    NB = 4
    assert b_per_w % (CH * NB) == 0
    n_chunks = b_per_w // CH
    groups = n_chunks // NB

    mesh = plsc.VectorSubcoreMesh(core_axis_name="c", subcore_axis_name="s")

    @functools.partial(
        pl.kernel,
        mesh=mesh,
        out_type=jax.ShapeDtypeStruct((B, D), dtype),
        compiler_params=pltpu.CompilerParams(use_tc_tiling_on_sc=False),
        scratch_types=[
            pltpu.VMEM((b_per_w,), jnp.int32),
            pltpu.VMEM((NB, CH, D), dtype),
        ]
        + [pltpu.SemaphoreType.DMA] * (2 * NB),
    )
    def gather_kernel(table_hbm, idx_hbm, out_hbm, idx_v, rows_v, *sems):
        gsems, ssems = sems[:NB], sems[NB:]
        wid = lax.axis_index("s") * NC + lax.axis_index("c")
        base = wid * b_per_w

        def g_copy(g, b):
            off = pl.multiple_of(g * CH, CH)
            return pltpu.make_async_copy(
                table_hbm.at[idx_v.at[pl.ds(off, CH)]], rows_v.at[b], gsems[b]
            )

        def s_copy(g, b):
            off = pl.multiple_of(base + g * CH, CH)
            return pltpu.make_async_copy(
                rows_v.at[b], out_hbm.at[pl.ds(off, CH)], ssems[b]
            )

        pltpu.sync_copy(idx_hbm.at[pl.ds(base, b_per_w)], idx_v)

        for b in range(NB):
            g_copy(b, b).start()

        def body(k, carry):
            for b in range(NB):
                g = k * NB + b
                g_copy(g, b).wait()
                s_copy(g, b).start()

                @pl.when(k < groups - 1)
                def _():
                    s_copy(g, b).wait()
                    g_copy(g + NB, b).start()

            return carry

        lax.fori_loop(0, groups, body, 0)

        for b in range(NB):
            s_copy((groups - 1) * NB + b, b).wait()

    return gather_kernel


def kernel(data, ivectors_weight):
    V, D = ivectors_weight.shape
    B = data.size
    idx = data.reshape(B).astype(jnp.int32)
    tb = ivectors_weight.astype(jnp.bfloat16)
    out = _build_gather(V, D, B, jnp.bfloat16)(tb, idx)
    return out.astype(jnp.float32).reshape(data.shape + (D,))


# speedup vs baseline: 1.7958x; 1.7958x over previous
import functools

import jax
import jax.numpy as jnp
from jax import lax
from jax.experimental import pallas as pl
from jax.experimental.pallas import tpu as pltpu
from jax.experimental.pallas import tpu_sc as plsc


@functools.lru_cache(maxsize=None)
def _build_gather(V, D, B, dtype):
    info = plsc.get_sparse_core_info()
    NC, NS = info.num_cores, info.num_subcores
    NW = NC * NS
    assert B % NW == 0
    b_per_w = B // NW
    CH = 256
    NB = 4
    assert b_per_w % (CH * NB) == 0
    n_chunks = b_per_w // CH
    groups = n_chunks // NB

    mesh = plsc.VectorSubcoreMesh(core_axis_name="c", subcore_axis_name="s")

    @functools.partial(
        pl.kernel,
        mesh=mesh,
        out_type=jax.ShapeDtypeStruct((B, D), dtype),
        compiler_params=pltpu.CompilerParams(use_tc_tiling_on_sc=False),
        scratch_types=[
            pltpu.VMEM((b_per_w,), jnp.int32),
            pltpu.VMEM((NB, CH, D), dtype),
        ]
        + [pltpu.SemaphoreType.DMA] * (2 * NB),
    )
    def gather_kernel(table_hbm, idx_hbm, out_hbm, idx_v, rows_v, *sems):
        gsems, ssems = sems[:NB], sems[NB:]
        wid = lax.axis_index("s") * NC + lax.axis_index("c")
        base = wid * b_per_w

        def g_copy(g, b):
            off = pl.multiple_of(g * CH, CH)
            return pltpu.make_async_copy(
                table_hbm.at[idx_v.at[pl.ds(off, CH)]], rows_v.at[b], gsems[b]
            )

        def s_copy(g, b):
            off = pl.multiple_of(base + g * CH, CH)
            return pltpu.make_async_copy(
                rows_v.at[b], out_hbm.at[pl.ds(off, CH)], ssems[b]
            )

        pltpu.sync_copy(idx_hbm.at[pl.ds(base, b_per_w)], idx_v)

        for b in range(NB):
            g_copy(b, b).start()

        def body(k, carry):
            for b in range(NB):
                g = k * NB + b
                g_copy(g, b).wait()
                s_copy(g, b).start()

                @pl.when(k < groups - 1)
                def _():
                    s_copy(g, b).wait()
                    g_copy(g + NB, b).start()

            return carry

        lax.fori_loop(0, groups, body, 0)

        for b in range(NB):
            s_copy((groups - 1) * NB + b, b).wait()

    return gather_kernel


def kernel(data, ivectors_weight):
    V, D = ivectors_weight.shape
    B = data.size
    idx = data.reshape(B).astype(jnp.int32)
    out = _build_gather(V, D, B, jnp.float32)(ivectors_weight, idx)
    return out.reshape(data.shape + (D,))
